# baseline (speedup 1.0000x reference)
import functools

import jax
import jax.numpy as jnp
from jax import lax
from jax.experimental import pallas as pl
from jax.experimental.pallas import tpu as pltpu
from jax.experimental.pallas import tpu_sc as plsc

Q = 4096
DIM = 64
NCLS = 64
BLK = 128
NC, NS, L = 2, 16, 16
NW = NC * NS
RPW = Q // NW
SLOT = 128
NCH = Q // L
HPAD = 144


def _tc_body(x_ref, xt_ref, labc_ref, labr_ref, d_ref, cnt_ref):
    i = pl.program_id(0)
    x = x_ref[...]
    xt = xt_ref[...]
    c = jnp.dot(x, xt, preferred_element_type=jnp.float32)
    s1 = jnp.sum(x * x, axis=1, keepdims=True)
    s2 = jnp.sum(xt * xt, axis=0, keepdims=True)
    dblk = jnp.maximum(s1 - 2.0 * c + s2, 0.0)
    d_ref[...] = dblk
    labr = labr_ref[...]
    labc = labc_ref[...]
    colid = lax.broadcasted_iota(jnp.int32, (BLK, Q), 1)
    rowid = i * BLK + lax.broadcasted_iota(jnp.int32, (BLK, Q), 0)
    same = (labr == labc) & (colid != rowid)
    dmin = jnp.min(jnp.where(same, dblk, jnp.inf), axis=1, keepdims=True)
    r1 = jnp.sum((dblk < dmin).astype(jnp.float32), axis=1, keepdims=True)
    part = jnp.sum((r1 <= colid.astype(jnp.float32)).astype(jnp.float32),
                   axis=0, keepdims=True)

    @pl.when(i == 0)
    def _():
        cnt_ref[...] = part

    @pl.when(i > 0)
    def _():
        cnt_ref[...] = cnt_ref[...] + part


def _tc_distance_ranks(x, labels):
    xt = x.T
    labc = labels.reshape(Q, 1)
    labr = labels.reshape(1, Q)
    grid = (Q // BLK,)
    return pl.pallas_call(
        _tc_body,
        grid=grid,
        in_specs=[
            pl.BlockSpec((BLK, DIM), lambda i: (i, 0)),
            pl.BlockSpec((DIM, Q), lambda i: (0, 0)),
            pl.BlockSpec((BLK, 1), lambda i: (i, 0)),
            pl.BlockSpec((1, Q), lambda i: (0, 0)),
        ],
        out_specs=[
            pl.BlockSpec((BLK, Q), lambda i: (i, 0)),
            pl.BlockSpec((1, Q), lambda i: (0, 0)),
        ],
        out_shape=[
            jax.ShapeDtypeStruct((Q, Q), jnp.float32),
            jax.ShapeDtypeStruct((1, Q), jnp.float32),
        ],
    )(x, xt, labc, labr)


def _lane_extract_f(v, l):
    io = lax.iota(jnp.int32, L)
    return jnp.sum(jnp.where(io == l, v, jnp.float32(0.0)))


def _lane_extract_i(v, l):
    io = lax.iota(jnp.int32, L)
    return jnp.sum(jnp.where(io == l, v, jnp.int32(0)))


def _count_scan(ref, nchunks, bcs, strict):
    def body(ck, accs):
        vals = ref[pl.ds(ck * L, L)]
        out = []
        for l in range(L):
            m = (vals < bcs[l]) if strict else (vals <= bcs[l])
            out.append(accs[l] + plsc.all_reduce_population_count(m))
        return tuple(out)

    accs = lax.fori_loop(0, nchunks, body,
                         tuple(jnp.zeros((L,), jnp.int32) for _ in range(L)))
    io = lax.iota(jnp.int32, L)
    res = jnp.zeros((L,), jnp.int32)
    for l in range(L):
        res = jnp.where(io == l, accs[l], res)
    return res


def _sc_body(d_hbm, lab_hbm, out_hbm,
             lab_v, lists_v, cnts_v, offs_v, row_v, row2_v, thr_v, sthr_v,
             thrrk_v, hist_v, pref_v, acc_v, sem, sem2):
    wid = lax.axis_index("s") * NC + lax.axis_index("c")
    io = lax.iota(jnp.int32, L)
    ones = jnp.ones((L,), jnp.int32)

    pltpu.sync_copy(lab_hbm, lab_v)

    def zero_body(k, _):
        cnts_v[pl.ds(k * L, L)] = jnp.zeros((L,), jnp.int32)
        return 0
    lax.fori_loop(0, NCLS * L // L, zero_body, 0)

    def hist_body(k, _):
        labs = lab_v[pl.ds(k * L, L)]
        plsc.addupdate_scatter(cnts_v, [labs * L + io], ones)
        return 0
    lax.fori_loop(0, NCH, hist_body, 0)

    def offs_body(cc, _):
        v = cnts_v[pl.ds(cc * L, L)]
        ex = plsc.cumsum(v) - v
        offs_v[pl.ds(cc * L, L)] = ex + cc * SLOT
        return 0
    lax.fori_loop(0, NCLS, offs_body, 0)

    def scat_body(k, _):
        labs = lab_v[pl.ds(k * L, L)]
        idx = labs * L + io
        pos = plsc.load_gather(offs_v, [idx])
        msk = pos < (labs + 1) * SLOT
        plsc.store_scatter(lists_v, [pos], k * L + io, mask=msk)
        plsc.addupdate_scatter(offs_v, [idx], ones, mask=msk)
        return 0
    lax.fori_loop(0, NCH, scat_body, 0)

    inf_v = jnp.full((L,), jnp.float32(jnp.inf))
    ioff = io * HPAD

    def process_row(row_v, i, acc):
        cchunk = lab_v[pl.ds((i // L) * L, L)]
        cls = _lane_extract_i(cchunk, i % L)
        ccnts = cnts_v[pl.ds(cls * L, L)]
        m = jnp.minimum(jnp.sum(ccnts), SLOT)
        ng = (m + (L - 1)) // L
        base = cls * SLOT
        mvec = jnp.full((L,), m - 1).astype(jnp.float32)
        inv_m = jnp.where(m > 1, jnp.ones((L,), jnp.float32) / mvec,
                          jnp.zeros((L,), jnp.float32))

        for g in range(SLOT // L):
            lane_k = g * L + io
            valid = lane_k < m
            idxs = lists_v[pl.ds(base + g * L, L)]
            sidx = jnp.where(valid, idxs, 0)
            tvals = plsc.load_gather(row_v, [sidx])
            thr_v[pl.ds(g * L, L)] = jnp.where(valid, tvals, inf_v)
            sthr_v[pl.ds(g * L, L)] = inf_v

        def rank_grp(g, _):
            tv = thr_v[pl.ds(g * L, L)]
            lane_k = g * L + io
            bcs = [plsc.load_gather(thr_v, [jnp.full((L,), g * L + l)])
                   for l in range(L)]
            rank = _count_scan(thr_v, ng, bcs, True)
            thrrk_v[pl.ds(g * L, L)] = rank
            plsc.store_scatter(sthr_v, [rank], tv, mask=lane_k < m)
            return 0
        lax.fori_loop(0, ng, rank_grp, 0)

        for c in range(L * HPAD // L):
            hist_v[pl.ds(c * L, L)] = jnp.zeros((L,), jnp.int32)

        UNR = 8
        pv = {q: plsc.load_gather(sthr_v, [jnp.full((L,), q)])
              for q in (15, 31, 47, 63, 79, 95, 111)}

        def srch(kk, _):
            vals_l = [row_v[pl.ds((kk * UNR + u) * L, L)] for u in range(UNR)]
            pb_l = []
            for v in vals_l:
                b1 = jnp.where(pv[63] < v, jnp.int32(64), jnp.int32(0))
                p2 = jnp.where(b1 > 0, pv[95], pv[31])
                b2 = jnp.where(p2 < v, b1 + 32, b1)
                p3a = jnp.where(b2 > 64, pv[111], pv[79])
                p3b = jnp.where(b2 > 0, pv[47], pv[15])
                p3 = jnp.where(b2 > 32, p3a, p3b)
                pb_l.append(jnp.where(p3 < v, b2 + 16, b2))
            b_l = pb_l
            for sstep in (8, 4, 2, 1):
                probe_l = [plsc.load_gather(sthr_v, [b + (sstep - 1)])
                           for b in b_l]
                b_l = [jnp.where(p < v, b + sstep, b)
                       for p, v, b in zip(probe_l, vals_l, b_l)]
            for u in range(UNR):
                plsc.addupdate_scatter(hist_v, [ioff + b_l[u]], ones)
            return 0
        lax.fori_loop(0, NCH // UNR, srch, 0)

        tots = []
        for c in range(HPAD // L):
            tot = hist_v[pl.ds(c * L, L)]
            for ll in range(1, L):
                tot = tot + hist_v[pl.ds(ll * HPAD + c * L, L)]
            tots.append(tot)
        sums = [jnp.sum(t) for t in tots]
        cums = [plsc.cumsum(t) for t in tots]
        carry = jnp.int32(0)
        for c in range(HPAD // L):
            pref_v[pl.ds(c * L, L)] = cums[c] + jnp.full((L,), carry)
            carry = carry + sums[c]

        def grp_body(g, acc_in):
            lane_k = g * L + io
            idxs = lists_v[pl.ds(base + g * L, L)]
            validj = (lane_k < m) & (idxs != i)
            rk = thrrk_v[pl.ds(g * L, L)]
            rho = plsc.load_gather(pref_v, [rk]) - 1
            contrib = rk.astype(jnp.float32) / rho.astype(jnp.float32)
            contrib = jnp.where(validj, contrib, jnp.float32(0.0))
            return acc_in + inv_m * contrib

        return lax.fori_loop(0, ng, grp_body, acc)

    row0 = wid * RPW
    pltpu.async_copy(d_hbm.at[row0], row_v, sem)

    def row_pair(rr, acc):
        i0 = row0 + 2 * rr
        i1 = i0 + 1
        pltpu.async_copy(d_hbm.at[i1], row2_v, sem2)
        pltpu.make_async_copy(d_hbm.at[i0], row_v, sem).wait()
        acc = process_row(row_v, i0, acc)
        inext = jnp.minimum(i0 + 2, row0 + RPW - 1)
        pltpu.async_copy(d_hbm.at[inext], row_v, sem)
        pltpu.make_async_copy(d_hbm.at[i1], row2_v, sem2).wait()
        return process_row(row2_v, i1, acc)

    acc = lax.fori_loop(0, RPW // 2, row_pair, jnp.zeros((L,), jnp.float32))
    pltpu.make_async_copy(d_hbm.at[row0], row_v, sem).wait()
    acc_v[...] = acc
    pltpu.sync_copy(acc_v, out_hbm.at[wid])


@functools.cache
def _make_sc_map():
    @functools.partial(
        pl.kernel,
        out_type=jax.ShapeDtypeStruct((NW, L), jnp.float32),
        scratch_types=[
            pltpu.VMEM((Q,), jnp.int32),
            pltpu.VMEM((NCLS * SLOT,), jnp.int32),
            pltpu.VMEM((NCLS * L,), jnp.int32),
            pltpu.VMEM((NCLS * L,), jnp.int32),
            pltpu.VMEM((Q,), jnp.float32),
            pltpu.VMEM((Q,), jnp.float32),
            pltpu.VMEM((SLOT,), jnp.float32),
            pltpu.VMEM((SLOT,), jnp.float32),
            pltpu.VMEM((SLOT,), jnp.int32),
            pltpu.VMEM((L * HPAD,), jnp.int32),
            pltpu.VMEM((HPAD,), jnp.int32),
            pltpu.VMEM((L,), jnp.float32),
            pltpu.SemaphoreType.DMA,
            pltpu.SemaphoreType.DMA,
        ],
        mesh=plsc.VectorSubcoreMesh(core_axis_name="c", subcore_axis_name="s"),
        compiler_params=pltpu.CompilerParams(needs_layout_passes=False),
    )
    def _sc_map(d_hbm, lab_hbm, out_hbm, *rest):
        _sc_body(d_hbm, lab_hbm, out_hbm, *rest)

    return _sc_map


def kernel(descriptors, test_labels):
    labels = test_labels.astype(jnp.int32)
    dmat, cnt = _tc_distance_ranks(descriptors, labels)
    ranks = cnt[0, 1:] * jnp.float32(1.0 / Q)
    partials = _make_sc_map()(dmat, labels)
    m_ap = jnp.sum(partials) * jnp.float32(1.0 / Q)
    return ranks, m_ap


# speedup vs baseline: 1.0371x; 1.0371x over previous
import functools

import jax
import jax.numpy as jnp
from jax import lax
from jax.experimental import pallas as pl
from jax.experimental.pallas import tpu as pltpu
from jax.experimental.pallas import tpu_sc as plsc

Q = 4096
DIM = 64
NCLS = 64
BLK = 128
NC, NS, L = 2, 16, 16
NW = NC * NS
RPW = Q // NW
SLOT = 128
NCH = Q // L
HPAD = 144


def _tc_body(x_ref, xt_ref, labc_ref, labr_ref, d_ref, cnt_ref):
    i = pl.program_id(0)
    x = x_ref[...]
    xt = xt_ref[...]
    c = jnp.dot(x, xt, preferred_element_type=jnp.float32)
    s1 = jnp.sum(x * x, axis=1, keepdims=True)
    s2 = jnp.sum(xt * xt, axis=0, keepdims=True)
    dblk = jnp.maximum(s1 - 2.0 * c + s2, 0.0)
    d_ref[...] = dblk
    labr = labr_ref[...]
    labc = labc_ref[...]
    colid = lax.broadcasted_iota(jnp.int32, (BLK, Q), 1)
    rowid = i * BLK + lax.broadcasted_iota(jnp.int32, (BLK, Q), 0)
    same = (labr == labc) & (colid != rowid)
    dmin = jnp.min(jnp.where(same, dblk, jnp.inf), axis=1, keepdims=True)
    r1 = jnp.sum((dblk < dmin).astype(jnp.float32), axis=1, keepdims=True)
    part = jnp.sum((r1 <= colid.astype(jnp.float32)).astype(jnp.float32),
                   axis=0, keepdims=True)

    @pl.when(i == 0)
    def _():
        cnt_ref[...] = part

    @pl.when(i > 0)
    def _():
        cnt_ref[...] = cnt_ref[...] + part


def _tc_distance_ranks(x, labels):
    xt = x.T
    labc = labels.reshape(Q, 1)
    labr = labels.reshape(1, Q)
    grid = (Q // BLK,)
    return pl.pallas_call(
        _tc_body,
        grid=grid,
        in_specs=[
            pl.BlockSpec((BLK, DIM), lambda i: (i, 0)),
            pl.BlockSpec((DIM, Q), lambda i: (0, 0)),
            pl.BlockSpec((BLK, 1), lambda i: (i, 0)),
            pl.BlockSpec((1, Q), lambda i: (0, 0)),
        ],
        out_specs=[
            pl.BlockSpec((BLK, Q), lambda i: (i, 0)),
            pl.BlockSpec((1, Q), lambda i: (0, 0)),
        ],
        out_shape=[
            jax.ShapeDtypeStruct((Q, Q), jnp.float32),
            jax.ShapeDtypeStruct((1, Q), jnp.float32),
        ],
    )(x, xt, labc, labr)


def _lane_extract_f(v, l):
    io = lax.iota(jnp.int32, L)
    return jnp.sum(jnp.where(io == l, v, jnp.float32(0.0)))


def _lane_extract_i(v, l):
    io = lax.iota(jnp.int32, L)
    return jnp.sum(jnp.where(io == l, v, jnp.int32(0)))


def _count_scan(ref, nchunks, thr, strict):
    bcs = [jnp.full((L,), _lane_extract_f(thr, l)) for l in range(L)]

    def body(ck, accs):
        vals = ref[pl.ds(ck * L, L)]
        out = []
        for l in range(L):
            m = (vals < bcs[l]) if strict else (vals <= bcs[l])
            out.append(accs[l] + plsc.all_reduce_population_count(m))
        return tuple(out)

    accs = lax.fori_loop(0, nchunks, body,
                         tuple(jnp.zeros((L,), jnp.int32) for _ in range(L)))
    io = lax.iota(jnp.int32, L)
    res = jnp.zeros((L,), jnp.int32)
    for l in range(L):
        res = jnp.where(io == l, accs[l], res)
    return res


def _sc_body(d_hbm, lab_hbm, out_hbm,
             lab_v, lists_v, cnts_v, offs_v, row_v, row2_v, thr_v, sthr_v,
             bins_v, hist_v, pref_v, acc_v, sem, sem2):
    wid = lax.axis_index("s") * NC + lax.axis_index("c")
    io = lax.iota(jnp.int32, L)
    ones = jnp.ones((L,), jnp.int32)

    pltpu.sync_copy(lab_hbm, lab_v)

    def zero_body(k, _):
        cnts_v[pl.ds(k * L, L)] = jnp.zeros((L,), jnp.int32)
        return 0
    lax.fori_loop(0, NCLS * L // L, zero_body, 0)

    def hist_body(k, _):
        labs = lab_v[pl.ds(k * L, L)]
        plsc.addupdate_scatter(cnts_v, [labs * L + io], ones)
        return 0
    lax.fori_loop(0, NCH, hist_body, 0)

    def offs_body(cc, _):
        v = cnts_v[pl.ds(cc * L, L)]
        ex = plsc.cumsum(v) - v
        offs_v[pl.ds(cc * L, L)] = ex + cc * SLOT
        return 0
    lax.fori_loop(0, NCLS, offs_body, 0)

    def scat_body(k, _):
        labs = lab_v[pl.ds(k * L, L)]
        idx = labs * L + io
        pos = plsc.load_gather(offs_v, [idx])
        msk = pos < (labs + 1) * SLOT
        plsc.store_scatter(lists_v, [pos], k * L + io, mask=msk)
        plsc.addupdate_scatter(offs_v, [idx], ones, mask=msk)
        return 0
    lax.fori_loop(0, NCH, scat_body, 0)

    inf_v = jnp.full((L,), jnp.float32(jnp.inf))
    ioff = io * HPAD

    def process_row(row_v, i, acc):
        cchunk = lab_v[pl.ds((i // L) * L, L)]
        cls = _lane_extract_i(cchunk, i % L)
        ccnts = cnts_v[pl.ds(cls * L, L)]
        m = jnp.minimum(jnp.sum(ccnts), SLOT)
        ng = (m + (L - 1)) // L
        base = cls * SLOT
        mvec = jnp.full((L,), m - 1).astype(jnp.float32)
        inv_m = jnp.where(m > 1, jnp.ones((L,), jnp.float32) / mvec,
                          jnp.zeros((L,), jnp.float32))

        for g in range(SLOT // L):
            lane_k = g * L + io
            valid = lane_k < m
            idxs = lists_v[pl.ds(base + g * L, L)]
            sidx = jnp.where(valid, idxs, 0)
            tvals = plsc.load_gather(row_v, [sidx])
            thr_v[pl.ds(g * L, L)] = jnp.where(valid, tvals, inf_v)
            sthr_v[pl.ds(g * L, L)] = inf_v

        def rank_grp(g, _):
            tv = thr_v[pl.ds(g * L, L)]
            lane_k = g * L + io
            rank = _count_scan(thr_v, ng, tv, True)
            plsc.store_scatter(sthr_v, [rank], tv, mask=lane_k < m)
            return 0
        lax.fori_loop(0, ng, rank_grp, 0)

        for c in range(L * HPAD // L):
            hist_v[pl.ds(c * L, L)] = jnp.zeros((L,), jnp.int32)

        UNR = 8
        piv = {}
        for q in (15, 31, 47, 63, 79, 95, 111):
            ch = sthr_v[pl.ds((q // L) * L, L)]
            piv[q] = jnp.max(jnp.where(io == (q % L), ch,
                                       jnp.float32(-jnp.inf)))
        pv = {q: jnp.full((L,), piv[q]) for q in piv}

        def srch(kk, _):
            vals_l = [row_v[pl.ds((kk * UNR + u) * L, L)] for u in range(UNR)]
            pb_l = []
            for v in vals_l:
                b1 = jnp.where(pv[63] < v, jnp.int32(64), jnp.int32(0))
                p2 = jnp.where(b1 > 0, pv[95], pv[31])
                b2 = jnp.where(p2 < v, b1 + 32, b1)
                p3a = jnp.where(b2 > 64, pv[111], pv[79])
                p3b = jnp.where(b2 > 0, pv[47], pv[15])
                p3 = jnp.where(b2 > 32, p3a, p3b)
                pb_l.append(jnp.where(p3 < v, b2 + 16, b2))
            b_l = pb_l
            for sstep in (8, 4, 2, 1):
                probe_l = [plsc.load_gather(sthr_v, [b + (sstep - 1)])
                           for b in b_l]
                b_l = [jnp.where(p < v, b + sstep, b)
                       for p, v, b in zip(probe_l, vals_l, b_l)]
            for u in range(UNR):
                bins_v[pl.ds((kk * UNR + u) * L, L)] = b_l[u]
                plsc.addupdate_scatter(hist_v, [ioff + b_l[u]], ones)
            return 0
        lax.fori_loop(0, NCH // UNR, srch, 0)

        tots = []
        for c in range(HPAD // L):
            tot = hist_v[pl.ds(c * L, L)]
            for ll in range(1, L):
                tot = tot + hist_v[pl.ds(ll * HPAD + c * L, L)]
            tots.append(tot)
        sums = [jnp.sum(t) for t in tots]
        cums = [plsc.cumsum(t) for t in tots]
        carry = jnp.int32(0)
        for c in range(HPAD // L):
            pref_v[pl.ds(c * L, L)] = cums[c] + jnp.full((L,), carry)
            carry = carry + sums[c]

        def grp_body(g, acc_in):
            lane_k = g * L + io
            idxs = lists_v[pl.ds(base + g * L, L)]
            validj = (lane_k < m) & (idxs != i)
            sidx = jnp.where(lane_k < m, idxs, 0)
            rk = plsc.load_gather(bins_v, [sidx])
            rho = plsc.load_gather(pref_v, [rk]) - 1
            contrib = rk.astype(jnp.float32) / rho.astype(jnp.float32)
            contrib = jnp.where(validj, contrib, jnp.float32(0.0))
            return acc_in + inv_m * contrib

        return lax.fori_loop(0, ng, grp_body, acc)

    row0 = wid * RPW
    pltpu.async_copy(d_hbm.at[row0], row_v, sem)

    def row_pair(rr, acc):
        i0 = row0 + 2 * rr
        i1 = i0 + 1
        pltpu.async_copy(d_hbm.at[i1], row2_v, sem2)
        pltpu.make_async_copy(d_hbm.at[i0], row_v, sem).wait()
        acc = process_row(row_v, i0, acc)
        inext = jnp.minimum(i0 + 2, row0 + RPW - 1)
        pltpu.async_copy(d_hbm.at[inext], row_v, sem)
        pltpu.make_async_copy(d_hbm.at[i1], row2_v, sem2).wait()
        return process_row(row2_v, i1, acc)

    acc = lax.fori_loop(0, RPW // 2, row_pair, jnp.zeros((L,), jnp.float32))
    pltpu.make_async_copy(d_hbm.at[row0], row_v, sem).wait()
    acc_v[...] = acc
    pltpu.sync_copy(acc_v, out_hbm.at[wid])


@functools.cache
def _make_sc_map():
    @functools.partial(
        pl.kernel,
        out_type=jax.ShapeDtypeStruct((NW, L), jnp.float32),
        scratch_types=[
            pltpu.VMEM((Q,), jnp.int32),
            pltpu.VMEM((NCLS * SLOT,), jnp.int32),
            pltpu.VMEM((NCLS * L,), jnp.int32),
            pltpu.VMEM((NCLS * L,), jnp.int32),
            pltpu.VMEM((Q,), jnp.float32),
            pltpu.VMEM((Q,), jnp.float32),
            pltpu.VMEM((SLOT,), jnp.float32),
            pltpu.VMEM((SLOT,), jnp.float32),
            pltpu.VMEM((Q,), jnp.int32),
            pltpu.VMEM((L * HPAD,), jnp.int32),
            pltpu.VMEM((HPAD,), jnp.int32),
            pltpu.VMEM((L,), jnp.float32),
            pltpu.SemaphoreType.DMA,
            pltpu.SemaphoreType.DMA,
        ],
        mesh=plsc.VectorSubcoreMesh(core_axis_name="c", subcore_axis_name="s"),
        compiler_params=pltpu.CompilerParams(needs_layout_passes=False),
    )
    def _sc_map(d_hbm, lab_hbm, out_hbm, *rest):
        _sc_body(d_hbm, lab_hbm, out_hbm, *rest)

    return _sc_map


def kernel(descriptors, test_labels):
    labels = test_labels.astype(jnp.int32)
    dmat, cnt = _tc_distance_ranks(descriptors, labels)
    ranks = cnt[0, 1:] * jnp.float32(1.0 / Q)
    partials = _make_sc_map()(dmat, labels)
    m_ap = jnp.sum(partials) * jnp.float32(1.0 / Q)
    return ranks, m_ap


# speedup vs baseline: 1.0640x; 1.0259x over previous
import functools

import jax
import jax.numpy as jnp
from jax import lax
from jax.experimental import pallas as pl
from jax.experimental.pallas import tpu as pltpu
from jax.experimental.pallas import tpu_sc as plsc

Q = 4096
DIM = 64
NCLS = 64
BLK = 128
NC, NS, L = 2, 16, 16
NW = NC * NS
RPW = Q // NW
SLOT = 128
NCH = Q // L
HPAD = 144


def _tc_body(x_ref, xt_ref, labc_ref, labr_ref, d_ref, cnt_ref):
    i = pl.program_id(0)
    x = x_ref[...]
    xt = xt_ref[...]
    c = jnp.dot(x, xt, preferred_element_type=jnp.float32)
    s1 = jnp.sum(x * x, axis=1, keepdims=True)
    s2 = jnp.sum(xt * xt, axis=0, keepdims=True)
    dblk = jnp.maximum(s1 - 2.0 * c + s2, 0.0)
    d_ref[...] = dblk
    labr = labr_ref[...]
    labc = labc_ref[...]
    colid = lax.broadcasted_iota(jnp.int32, (BLK, Q), 1)
    rowid = i * BLK + lax.broadcasted_iota(jnp.int32, (BLK, Q), 0)
    same = (labr == labc) & (colid != rowid)
    dmin = jnp.min(jnp.where(same, dblk, jnp.inf), axis=1, keepdims=True)
    r1 = jnp.sum((dblk < dmin).astype(jnp.float32), axis=1, keepdims=True)
    part = jnp.sum((r1 <= colid.astype(jnp.float32)).astype(jnp.float32),
                   axis=0, keepdims=True)

    @pl.when(i == 0)
    def _():
        cnt_ref[...] = part

    @pl.when(i > 0)
    def _():
        cnt_ref[...] = cnt_ref[...] + part


def _tc_distance_ranks(x, labels):
    xt = x.T
    labc = labels.reshape(Q, 1)
    labr = labels.reshape(1, Q)
    grid = (Q // BLK,)
    return pl.pallas_call(
        _tc_body,
        grid=grid,
        in_specs=[
            pl.BlockSpec((BLK, DIM), lambda i: (i, 0)),
            pl.BlockSpec((DIM, Q), lambda i: (0, 0)),
            pl.BlockSpec((BLK, 1), lambda i: (i, 0)),
            pl.BlockSpec((1, Q), lambda i: (0, 0)),
        ],
        out_specs=[
            pl.BlockSpec((BLK, Q), lambda i: (i, 0)),
            pl.BlockSpec((1, Q), lambda i: (0, 0)),
        ],
        out_shape=[
            jax.ShapeDtypeStruct((Q, Q), jnp.float32),
            jax.ShapeDtypeStruct((1, Q), jnp.float32),
        ],
    )(x, xt, labc, labr)


def _lane_extract_f(v, l):
    io = lax.iota(jnp.int32, L)
    return jnp.sum(jnp.where(io == l, v, jnp.float32(0.0)))


def _lane_extract_i(v, l):
    io = lax.iota(jnp.int32, L)
    return jnp.sum(jnp.where(io == l, v, jnp.int32(0)))


def _count_scan(ref, nchunks, thr, strict):
    bcs = [jnp.full((L,), _lane_extract_f(thr, l)) for l in range(L)]

    def body(ck, accs):
        vals = ref[pl.ds(ck * L, L)]
        out = []
        for l in range(L):
            m = (vals < bcs[l]) if strict else (vals <= bcs[l])
            out.append(accs[l] + plsc.all_reduce_population_count(m))
        return tuple(out)

    accs = lax.fori_loop(0, nchunks, body,
                         tuple(jnp.zeros((L,), jnp.int32) for _ in range(L)))
    io = lax.iota(jnp.int32, L)
    res = jnp.zeros((L,), jnp.int32)
    for l in range(L):
        res = jnp.where(io == l, accs[l], res)
    return res


def _sc_body(d_hbm, lab_hbm, out_hbm,
             lab_v, lists_v, cnts_v, offs_v, row_v, row2_v, thr_v, sthr_v,
             bins_v, hist_v, pref_v, acc_v, sem, sem2):
    wid = lax.axis_index("s") * NC + lax.axis_index("c")
    io = lax.iota(jnp.int32, L)
    ones = jnp.ones((L,), jnp.int32)

    pltpu.sync_copy(lab_hbm, lab_v)

    def zero_body(k, _):
        cnts_v[pl.ds(k * L, L)] = jnp.zeros((L,), jnp.int32)
        return 0
    lax.fori_loop(0, NCLS * L // L, zero_body, 0)

    def hist_body(k, _):
        labs = lab_v[pl.ds(k * L, L)]
        plsc.addupdate_scatter(cnts_v, [labs * L + io], ones)
        return 0
    lax.fori_loop(0, NCH, hist_body, 0)

    def offs_body(cc, _):
        v = cnts_v[pl.ds(cc * L, L)]
        ex = plsc.cumsum(v) - v
        offs_v[pl.ds(cc * L, L)] = ex + cc * SLOT
        return 0
    lax.fori_loop(0, NCLS, offs_body, 0)

    def scat_body(k, _):
        labs = lab_v[pl.ds(k * L, L)]
        idx = labs * L + io
        pos = plsc.load_gather(offs_v, [idx])
        msk = pos < (labs + 1) * SLOT
        plsc.store_scatter(lists_v, [pos], k * L + io, mask=msk)
        plsc.addupdate_scatter(offs_v, [idx], ones, mask=msk)
        return 0
    lax.fori_loop(0, NCH, scat_body, 0)

    inf_v = jnp.full((L,), jnp.float32(jnp.inf))
    ioff = io * HPAD

    def process_row(row_v, i, acc):
        cchunk = lab_v[pl.ds((i // L) * L, L)]
        cls = _lane_extract_i(cchunk, i % L)
        ccnts = cnts_v[pl.ds(cls * L, L)]
        m = jnp.minimum(jnp.sum(ccnts), SLOT)
        ng = (m + (L - 1)) // L
        base = cls * SLOT
        mvec = jnp.full((L,), m - 1).astype(jnp.float32)
        inv_m = jnp.where(m > 1, jnp.ones((L,), jnp.float32) / mvec,
                          jnp.zeros((L,), jnp.float32))

        for g in range(SLOT // L):
            lane_k = g * L + io
            valid = lane_k < m
            idxs = lists_v[pl.ds(base + g * L, L)]
            sidx = jnp.where(valid, idxs, 0)
            tvals = plsc.load_gather(row_v, [sidx])
            thr_v[pl.ds(g * L, L)] = jnp.where(valid, tvals, inf_v)
            sthr_v[pl.ds(g * L, L)] = inf_v

        def rank_grp(g, _):
            tv = thr_v[pl.ds(g * L, L)]
            lane_k = g * L + io
            rank = _count_scan(thr_v, ng, tv, True)
            plsc.store_scatter(sthr_v, [rank], tv, mask=lane_k < m)
            return 0
        lax.fori_loop(0, ng, rank_grp, 0)

        zvec = jnp.zeros((L,), jnp.int32)

        def zero_hist(c, _):
            for ll in range(L):
                hist_v[pl.ds(ll * HPAD + c * L, L)] = zvec
            return 0
        lax.fori_loop(0, (m + 2 + L - 1) // L, zero_hist, 0)

        UNR = 8
        piv = {}
        for q in (15, 31, 47, 63, 79, 95, 111):
            ch = sthr_v[pl.ds((q // L) * L, L)]
            piv[q] = jnp.max(jnp.where(io == (q % L), ch,
                                       jnp.float32(-jnp.inf)))
        pv = {q: jnp.full((L,), piv[q]) for q in piv}

        def srch(kk, _):
            vals_l = [row_v[pl.ds((kk * UNR + u) * L, L)] for u in range(UNR)]
            pb_l = []
            for v in vals_l:
                b1 = jnp.where(pv[63] < v, jnp.int32(64), jnp.int32(0))
                p2 = jnp.where(b1 > 0, pv[95], pv[31])
                b2 = jnp.where(p2 < v, b1 + 32, b1)
                p3a = jnp.where(b2 > 64, pv[111], pv[79])
                p3b = jnp.where(b2 > 0, pv[47], pv[15])
                p3 = jnp.where(b2 > 32, p3a, p3b)
                pb_l.append(jnp.where(p3 < v, b2 + 16, b2))
            b_l = pb_l
            for sstep in (8, 4, 2, 1):
                probe_l = [plsc.load_gather(sthr_v, [b + (sstep - 1)])
                           for b in b_l]
                b_l = [jnp.where(p < v, b + sstep, b)
                       for p, v, b in zip(probe_l, vals_l, b_l)]
            for u in range(UNR):
                bins_v[pl.ds((kk * UNR + u) * L, L)] = b_l[u]
                plsc.addupdate_scatter(hist_v, [ioff + b_l[u]], ones)
            return 0
        lax.fori_loop(0, NCH // UNR, srch, 0)

        tots = []
        for c in range(HPAD // L):
            tot = hist_v[pl.ds(c * L, L)]
            for ll in range(1, L):
                tot = tot + hist_v[pl.ds(ll * HPAD + c * L, L)]
            tots.append(tot)
        sums = [jnp.sum(t) for t in tots]
        cums = [plsc.cumsum(t) for t in tots]
        carry = jnp.int32(0)
        for c in range(HPAD // L):
            pref_v[pl.ds(c * L, L)] = cums[c] + jnp.full((L,), carry)
            carry = carry + sums[c]

        def grp_body(g, acc_in):
            lane_k = g * L + io
            idxs = lists_v[pl.ds(base + g * L, L)]
            validj = (lane_k < m) & (idxs != i)
            sidx = jnp.where(lane_k < m, idxs, 0)
            rk = plsc.load_gather(bins_v, [sidx])
            rho = plsc.load_gather(pref_v, [rk]) - 1
            contrib = rk.astype(jnp.float32) / rho.astype(jnp.float32)
            contrib = jnp.where(validj, contrib, jnp.float32(0.0))
            return acc_in + inv_m * contrib

        return lax.fori_loop(0, ng, grp_body, acc)

    row0 = wid * RPW
    pltpu.async_copy(d_hbm.at[row0], row_v, sem)

    def row_pair(rr, acc):
        i0 = row0 + 2 * rr
        i1 = i0 + 1
        pltpu.async_copy(d_hbm.at[i1], row2_v, sem2)
        pltpu.make_async_copy(d_hbm.at[i0], row_v, sem).wait()
        acc = process_row(row_v, i0, acc)
        inext = jnp.minimum(i0 + 2, row0 + RPW - 1)
        pltpu.async_copy(d_hbm.at[inext], row_v, sem)
        pltpu.make_async_copy(d_hbm.at[i1], row2_v, sem2).wait()
        return process_row(row2_v, i1, acc)

    acc = lax.fori_loop(0, RPW // 2, row_pair, jnp.zeros((L,), jnp.float32))
    pltpu.make_async_copy(d_hbm.at[row0], row_v, sem).wait()
    acc_v[...] = acc
    pltpu.sync_copy(acc_v, out_hbm.at[wid])


@functools.cache
def _make_sc_map():
    @functools.partial(
        pl.kernel,
        out_type=jax.ShapeDtypeStruct((NW, L), jnp.float32),
        scratch_types=[
            pltpu.VMEM((Q,), jnp.int32),
            pltpu.VMEM((NCLS * SLOT,), jnp.int32),
            pltpu.VMEM((NCLS * L,), jnp.int32),
            pltpu.VMEM((NCLS * L,), jnp.int32),
            pltpu.VMEM((Q,), jnp.float32),
            pltpu.VMEM((Q,), jnp.float32),
            pltpu.VMEM((SLOT,), jnp.float32),
            pltpu.VMEM((SLOT,), jnp.float32),
            pltpu.VMEM((Q,), jnp.int32),
            pltpu.VMEM((L * HPAD,), jnp.int32),
            pltpu.VMEM((HPAD,), jnp.int32),
            pltpu.VMEM((L,), jnp.float32),
            pltpu.SemaphoreType.DMA,
            pltpu.SemaphoreType.DMA,
        ],
        mesh=plsc.VectorSubcoreMesh(core_axis_name="c", subcore_axis_name="s"),
        compiler_params=pltpu.CompilerParams(needs_layout_passes=False),
    )
    def _sc_map(d_hbm, lab_hbm, out_hbm, *rest):
        _sc_body(d_hbm, lab_hbm, out_hbm, *rest)

    return _sc_map


def kernel(descriptors, test_labels):
    labels = test_labels.astype(jnp.int32)
    dmat, cnt = _tc_distance_ranks(descriptors, labels)
    ranks = cnt[0, 1:] * jnp.float32(1.0 / Q)
    partials = _make_sc_map()(dmat, labels)
    m_ap = jnp.sum(partials) * jnp.float32(1.0 / Q)
    return ranks, m_ap
